# Initial kernel scaffold; baseline (speedup 1.0000x reference)
import functools

import jax
import jax.numpy as jnp
from jax import lax
from jax.experimental import pallas as pl
from jax.experimental.pallas import tpu as pltpu
from jax.experimental.pallas import tpu_sc as plsc

N = 10000
E = 160000
F = 256

L = 16
NV = F // L
NC = 2
NS = 16
NW = NC * NS
CHUNK = 128
NCHUNKS = E // CHUNK


def _matmul_body(x_ref, w_ref, y_ref):
    y_ref[...] = jnp.dot(x_ref[...], w_ref[...],
                         preferred_element_type=jnp.float32)


def _tc_matmul(x, w):
    blk = 1000
    return pl.pallas_call(
        _matmul_body,
        grid=(N // blk,),
        in_specs=[
            pl.BlockSpec((blk, F), lambda i: (i, 0)),
            pl.BlockSpec((F, F), lambda i: (0, 0)),
        ],
        out_specs=pl.BlockSpec((blk, F), lambda i: (i, 0)),
        out_shape=jax.ShapeDtypeStruct((N, F), jnp.float32),
    )(x, w)


def _sc_edge_body(y_hbm, src_hbm, dst_hbm, d_hbm, pmax_hbm, cov_hbm,
                  src_v, dst_v, d_v, rows_s, rows_d, acc_v, ones_v, zero_v,
                  cov_sh, sem_s, sem_d):
    cid = lax.axis_index("c")
    sid = lax.axis_index("s")
    wid = cid * NS + sid

    for k in range(CHUNK // L):
        ones_v[pl.ds(k * L, L)] = jnp.ones((L,), jnp.int32)

    @pl.when(sid == 0)
    def _():
        def zbody(k, carry):
            zero_v[pl.ds(k * L, L)] = jnp.zeros((L,), jnp.int32)
            return carry
        lax.fori_loop(0, N // L, zbody, 0)
        pltpu.sync_copy(zero_v, cov_sh)

    plsc.subcore_barrier()

    nchunks_w = (NCHUNKS - wid + NW - 1) // NW

    def chunk_body(j, accs):
        c = wid + j * NW
        base = c * CHUNK
        pltpu.sync_copy(src_hbm.at[pl.ds(base, CHUNK)], src_v)
        pltpu.sync_copy(dst_hbm.at[pl.ds(base, CHUNK)], dst_v)
        pltpu.sync_copy(d_hbm.at[pl.ds(base, CHUNK)], d_v)
        cp_s = pltpu.async_copy(y_hbm.at[src_v], rows_s, sem_s)
        cp_d = pltpu.async_copy(y_hbm.at[dst_v], rows_d, sem_d)
        pltpu.sync_copy(ones_v, cov_sh.at[dst_v], add=True)
        cp_s.wait()
        cp_d.wait()

        def edge_body(i, accs):
            d_bcast = plsc.load_gather(d_v, [jnp.full((L,), i, jnp.int32)])
            new = []
            for f in range(NV):
                s = rows_s[i, pl.ds(f * L, L)] + rows_d[i, pl.ds(f * L, L)]
                new.append(jnp.maximum(accs[f], s * d_bcast))
            return tuple(new)

        return lax.fori_loop(0, CHUNK, edge_body, accs)

    accs = tuple(jnp.full((L,), -jnp.inf, jnp.float32) for _ in range(NV))
    accs = lax.fori_loop(0, nchunks_w, chunk_body, accs)

    for f in range(NV):
        acc_v[pl.ds(f * L, L)] = accs[f]
    pltpu.sync_copy(acc_v, pmax_hbm.at[wid])

    plsc.subcore_barrier()

    @pl.when(sid == 0)
    def _():
        pltpu.sync_copy(cov_sh, cov_hbm.at[cid])


_sc_edge = functools.partial(
    pl.kernel,
    mesh=plsc.VectorSubcoreMesh(core_axis_name="c", subcore_axis_name="s"),
    out_type=[
        jax.ShapeDtypeStruct((NW, F), jnp.float32),
        jax.ShapeDtypeStruct((NC, N), jnp.int32),
    ],
    scratch_types=[
        pltpu.VMEM((CHUNK,), jnp.int32),
        pltpu.VMEM((CHUNK,), jnp.int32),
        pltpu.VMEM((CHUNK,), jnp.float32),
        pltpu.VMEM((CHUNK, F), jnp.float32),
        pltpu.VMEM((CHUNK, F), jnp.float32),
        pltpu.VMEM((F,), jnp.float32),
        pltpu.VMEM((CHUNK,), jnp.int32),
        pltpu.VMEM((N,), jnp.int32),
        pltpu.VMEM_SHARED((N,), jnp.int32),
        pltpu.SemaphoreType.DMA,
        pltpu.SemaphoreType.DMA,
    ],
)(_sc_edge_body)


def _combine_body(pmax_ref, cov_ref, b_ref, out_ref):
    m = jnp.max(pmax_ref[...], axis=0, keepdims=True) + b_ref[...]
    indeg = cov_ref[0:1, :] + cov_ref[1:2, :]
    has_empty = jnp.min(indeg) == 0
    out_ref[...] = jnp.where(has_empty, jnp.maximum(m, 0.0), m)


def _tc_combine(pmax, cov, b):
    return pl.pallas_call(
        _combine_body,
        out_shape=jax.ShapeDtypeStruct((1, F), jnp.float32),
    )(pmax, cov, b)


@jax.jit
def kernel(x, edge_index, edge_d, theta_W, theta_b):
    src = edge_index[0].astype(jnp.int32)
    dst = edge_index[1].astype(jnp.int32)
    y = _tc_matmul(x, theta_W)
    pmax, cov = _sc_edge(y, src, dst, edge_d)
    return _tc_combine(pmax, cov, theta_b.reshape(1, F))


# speedup vs baseline: 6.7882x; 6.7882x over previous
import functools

import jax
import jax.numpy as jnp
from jax import lax
from jax.experimental import pallas as pl
from jax.experimental.pallas import tpu as pltpu
from jax.experimental.pallas import tpu_sc as plsc

N = 10000
E = 160000
F = 256

L = 16
NV = F // L
NC = 2
NS = 16
NW = NC * NS
CHUNK = 80
NCHUNKS = E // CHUNK
NCH_MAX = 64


def _matmul_body(x_ref, w_ref, y_ref):
    y_ref[...] = jnp.dot(x_ref[...], w_ref[...],
                         preferred_element_type=jnp.float32)


def _tc_matmul(x, w):
    blk = 1000
    return pl.pallas_call(
        _matmul_body,
        grid=(N // blk,),
        in_specs=[
            pl.BlockSpec((blk, F), lambda i: (i, 0)),
            pl.BlockSpec((F, F), lambda i: (0, 0)),
        ],
        out_specs=pl.BlockSpec((blk, F), lambda i: (i, 0)),
        out_shape=jax.ShapeDtypeStruct((N, F), jnp.float32),
    )(x, w)


def _sc_edge_body(y_hbm, idx_hbm, d_hbm, pmax_hbm, cov_hbm,
                  idx0, idx1, d0, d1, rs0, rs1, rd0, rd1, acc_v, ones_v,
                  zero_v, cov_sh,
                  sem_i0, sem_i1, sem_w0, sem_w1,
                  sem_s0, sem_s1, sem_d0, sem_d1):
    cid = lax.axis_index("c")
    sid = lax.axis_index("s")
    wid = cid * NS + sid

    for k in range(CHUNK // L):
        ones_v[pl.ds(k * L, L)] = jnp.ones((L,), jnp.int32)

    @pl.when(sid == 0)
    def _():
        def zbody(k, carry):
            zero_v[pl.ds(k * L, L)] = jnp.zeros((L,), jnp.int32)
            return carry
        lax.fori_loop(0, N // L, zbody, 0)
        pltpu.sync_copy(zero_v, cov_sh)

    plsc.subcore_barrier()

    bufs = (
        (idx0, d0, rs0, rd0, sem_i0, sem_w0, sem_s0, sem_d0),
        (idx1, d1, rs1, rd1, sem_i1, sem_w1, sem_s1, sem_d1),
    )

    def chunk_id(j):
        return wid + j * NW

    def issue_meta(j, b):
        c = chunk_id(j)
        idx_v, d_v = bufs[b][0], bufs[b][1]
        @pl.when(c < NCHUNKS)
        def _():
            pltpu.async_copy(idx_hbm.at[c], idx_v, bufs[b][4])
            pltpu.async_copy(d_hbm.at[c], d_v, bufs[b][5])

    def issue_rows(j, b):
        c = chunk_id(j)
        idx_v, d_v, rs, rd = bufs[b][0], bufs[b][1], bufs[b][2], bufs[b][3]
        @pl.when(c < NCHUNKS)
        def _():
            pltpu.make_async_copy(idx_hbm.at[c], idx_v, bufs[b][4]).wait()
            pltpu.make_async_copy(d_hbm.at[c], d_v, bufs[b][5]).wait()
            pltpu.async_copy(y_hbm.at[idx_v.at[0]], rs, bufs[b][6])
            pltpu.async_copy(y_hbm.at[idx_v.at[1]], rd, bufs[b][7])
            pltpu.sync_copy(ones_v, cov_sh.at[idx_v.at[1]], add=True)

    def consume(j, b, accs):
        c = chunk_id(j)
        valid = c < NCHUNKS
        idx_v, d_v, rs, rd = bufs[b][0], bufs[b][1], bufs[b][2], bufs[b][3]
        @pl.when(valid)
        def _():
            pltpu.make_async_copy(y_hbm.at[idx_v.at[0]], rs, bufs[b][6]).wait()
            pltpu.make_async_copy(y_hbm.at[idx_v.at[1]], rd, bufs[b][7]).wait()

        def edge_body(i, accs):
            d_bcast = d_v[i, pl.ds(0, L)]
            new = []
            for f in range(NV):
                s = rs[i, pl.ds(f * L, L)] + rd[i, pl.ds(f * L, L)]
                new.append(jnp.maximum(accs[f], s * d_bcast))
            return tuple(new)

        cand = lax.fori_loop(0, CHUNK, edge_body, accs)
        return tuple(jnp.where(valid, cn, ao) for cn, ao in zip(cand, accs))

    issue_meta(0, 0)
    issue_meta(1, 1)
    issue_rows(0, 0)

    def step(k, accs):
        for half in range(2):
            j = 2 * k + half
            b = half
            issue_rows(j + 1, 1 - b)
            accs = consume(j, b, accs)
            issue_meta(j + 2, b)
        return accs

    accs = tuple(jnp.full((L,), -jnp.inf, jnp.float32) for _ in range(NV))
    accs = lax.fori_loop(0, NCH_MAX // 2, step, accs)

    for f in range(NV):
        acc_v[pl.ds(f * L, L)] = accs[f]
    pltpu.sync_copy(acc_v, pmax_hbm.at[wid])

    plsc.subcore_barrier()

    @pl.when(sid == 0)
    def _():
        pltpu.sync_copy(cov_sh, cov_hbm.at[cid])


_sc_edge = functools.partial(
    pl.kernel,
    mesh=plsc.VectorSubcoreMesh(core_axis_name="c", subcore_axis_name="s"),
    out_type=[
        jax.ShapeDtypeStruct((NW, F), jnp.float32),
        jax.ShapeDtypeStruct((NC, N), jnp.int32),
    ],
    scratch_types=[
        pltpu.VMEM((2, CHUNK), jnp.int32),
        pltpu.VMEM((2, CHUNK), jnp.int32),
        pltpu.VMEM((CHUNK, L), jnp.float32),
        pltpu.VMEM((CHUNK, L), jnp.float32),
        pltpu.VMEM((CHUNK, F), jnp.float32),
        pltpu.VMEM((CHUNK, F), jnp.float32),
        pltpu.VMEM((CHUNK, F), jnp.float32),
        pltpu.VMEM((CHUNK, F), jnp.float32),
        pltpu.VMEM((F,), jnp.float32),
        pltpu.VMEM((CHUNK,), jnp.int32),
        pltpu.VMEM((N,), jnp.int32),
        pltpu.VMEM_SHARED((N,), jnp.int32),
        pltpu.SemaphoreType.DMA,
        pltpu.SemaphoreType.DMA,
        pltpu.SemaphoreType.DMA,
        pltpu.SemaphoreType.DMA,
        pltpu.SemaphoreType.DMA,
        pltpu.SemaphoreType.DMA,
        pltpu.SemaphoreType.DMA,
        pltpu.SemaphoreType.DMA,
    ],
)(_sc_edge_body)


def _combine_body(pmax_ref, cov_ref, b_ref, out_ref):
    m = jnp.max(pmax_ref[...], axis=0, keepdims=True) + b_ref[...]
    indeg = cov_ref[0:1, :] + cov_ref[1:2, :]
    has_empty = jnp.min(indeg) == 0
    out_ref[...] = jnp.where(has_empty, jnp.maximum(m, 0.0), m)


def _tc_combine(pmax, cov, b):
    return pl.pallas_call(
        _combine_body,
        out_shape=jax.ShapeDtypeStruct((1, F), jnp.float32),
    )(pmax, cov, b)


@jax.jit
def kernel(x, edge_index, edge_d, theta_W, theta_b):
    src = edge_index[0].astype(jnp.int32)
    dst = edge_index[1].astype(jnp.int32)
    y = _tc_matmul(x, theta_W)
    idx = jnp.stack(
        [src.reshape(NCHUNKS, CHUNK), dst.reshape(NCHUNKS, CHUNK)], axis=1)
    d_rep = jnp.broadcast_to(
        edge_d[:, None], (E, L)).reshape(NCHUNKS, CHUNK, L)
    pmax, cov = _sc_edge(y, idx, d_rep)
    return _tc_combine(pmax, cov, theta_b.reshape(1, F))
